# baseline (speedup 1.0000x reference)
import functools

import jax
import jax.numpy as jnp
from jax import lax
from jax.experimental import pallas as pl
from jax.experimental.pallas import tpu as pltpu
from jax.experimental.pallas import tpu_sc as plsc

N = 10000
NNZ = 160000
C = 256
G = 64
NCORES = 2
NTILES = 16
B = 128
NCHUNK = 80
PER_TILE = NCHUNK * B
NNZ_PAD = NTILES * PER_TILE
APAD = 10112
RPT = APAD // NTILES
NBUF = 2

_MESH = plsc.VectorSubcoreMesh(
    core_axis_name="c", subcore_axis_name="s",
    num_cores=NCORES, num_subcores=NTILES)


def _mm_body(x_ref, w_ref, b_ref, o0_ref, o1_ref, o2_ref, o3_ref):
    acc = jnp.dot(x_ref[...], w_ref[...],
                  preferred_element_type=jnp.float32) + b_ref[...]
    o0_ref[...] = acc[:, 0 * G:1 * G]
    o1_ref[...] = acc[:, 1 * G:2 * G]
    o2_ref[...] = acc[:, 2 * G:3 * G]
    o3_ref[...] = acc[:, 3 * G:4 * G]


def _matmul(x, w, b2):
    return pl.pallas_call(
        _mm_body,
        grid=(10,),
        in_specs=[
            pl.BlockSpec((1000, C), lambda i: (i, 0)),
            pl.BlockSpec((C, C), lambda i: (0, 0)),
            pl.BlockSpec((1, C), lambda i: (0, 0)),
        ],
        out_specs=[pl.BlockSpec((1000, G), lambda i: (i, 0))] * 4,
        out_shape=[jax.ShapeDtypeStruct((N, G), jnp.float32)] * 4,
    )(x, w, b2)


def _stream_loop(table, gidx_v, sidx_v, bufs, acc, cnt_add=None,
                 table_hbm=None):
    def _issue(chunk, rows, gsem):
        if table_hbm is None:
            pltpu.async_copy(table.at[gidx_v.at[chunk]], rows, gsem)
        elif isinstance(chunk, int):
            src = table_hbm if chunk % 4 == 1 else table
            pltpu.async_copy(src.at[gidx_v.at[chunk]], rows, gsem)
        else:
            m = chunk % 4

            @pl.when(m == 1)
            def _():
                pltpu.async_copy(table_hbm.at[gidx_v.at[chunk]], rows, gsem)

            @pl.when(m != 1)
            def _():
                pltpu.async_copy(table.at[gidx_v.at[chunk]], rows, gsem)

    for b, (rows, gsem, ssem) in enumerate(bufs):
        _issue(b, rows, gsem)

    @pl.loop(0, NCHUNK, step=NBUF)
    def _(j):
        for b, (rows, gsem, ssem) in enumerate(bufs):
            jj = j + b
            pltpu.make_async_copy(table.at[gidx_v.at[jj]], rows, gsem).wait()
            pltpu.async_copy(rows, acc.at[sidx_v.at[jj]], ssem, add=True)
            if cnt_add is not None:
                ones_v, cnt_acc = cnt_add
                pltpu.async_copy(ones_v, cnt_acc.at[sidx_v.at[jj]], ssem,
                                 add=True)

            @pl.when(jj + NBUF < NCHUNK)
            def _():
                pltpu.make_async_copy(rows, acc.at[sidx_v.at[jj]], ssem).wait()
                if cnt_add is not None:
                    ones_v, cnt_acc = cnt_add
                    pltpu.make_async_copy(
                        ones_v, cnt_acc.at[sidx_v.at[jj]], ssem).wait()
                _issue(jj + NBUF, rows, gsem)

    for b, (rows, gsem, ssem) in enumerate(bufs):
        jj = NCHUNK - NBUF + b
        pltpu.make_async_copy(rows, acc.at[sidx_v.at[jj]], ssem).wait()
        if cnt_add is not None:
            ones_v, cnt_acc = cnt_add
            pltpu.make_async_copy(ones_v, cnt_acc.at[sidx_v.at[jj]],
                                  ssem).wait()


def _stage_tbl(src_hbm, tbl, sid):
    @pl.when(sid < 10)
    def _():
        off = sid * 1000
        pltpu.sync_copy(src_hbm.at[pl.ds(off, 1000)], tbl.at[pl.ds(off, 1000)])


def _convert(acc, cnt_acc, tbl, rows0, ones_v, r0):
    for off, rows_n in ((0, B), (B, B), (2 * B, B), (3 * B, B), (4 * B, 120)):
        base = r0 + off
        pltpu.sync_copy(acc.at[pl.ds(base, rows_n)],
                        rows0.at[pl.ds(0, rows_n)])
        pltpu.sync_copy(cnt_acc.at[pl.ds(base, rows_n)],
                        ones_v.at[pl.ds(0, rows_n)])

        @pl.loop(0, rows_n)
        def _(r):
            inv = 1.0 / jnp.maximum(ones_v[r], 1.0)
            for k in range(G // 16):
                sl = pl.ds(k * 16, 16)
                rows0[r, sl] = rows0[r, sl] * inv

        pltpu.sync_copy(rows0.at[pl.ds(0, rows_n)],
                        tbl.at[pl.ds(base, rows_n)])


def _conv_body(t0, t1, t2, t3, vg_h, es_h, eg_h, vs_h, zrow, zcnt, ones_h,
               o0, o1, o2, o3,
               gidx_v, sidx_v, r0b, r1b, ones_v, tbl, acc, cnt_acc,
               g0, g1, s0m, s1m):
    bufs = [(r0b, g0, s0m), (r1b, g1, s1m)]
    cid = lax.axis_index("c")
    sid = lax.axis_index("s")
    r0 = sid * RPT

    for phase in (0, 1):
        pltpu.sync_copy(zrow, acc.at[pl.ds(r0, RPT)])
        pltpu.sync_copy(vg_h.at[sid], gidx_v)
        pltpu.sync_copy(es_h.at[sid], sidx_v)
        if phase == 0:
            pltpu.sync_copy(zcnt, cnt_acc.at[pl.ds(r0, RPT)])
            pltpu.sync_copy(ones_h, ones_v)

        @pl.when(cid == 0)
        def _():
            _stage_tbl(t0 if phase == 0 else t1, tbl, sid)

        @pl.when(cid == 1)
        def _():
            _stage_tbl(t2 if phase == 0 else t3, tbl, sid)

        plsc.subcore_barrier()

        cadd = (ones_v, cnt_acc) if phase == 0 else None

        @pl.when(cid == 0)
        def _():
            _stream_loop(tbl, gidx_v, sidx_v, bufs, acc, cnt_add=cadd,
                         table_hbm=t0 if phase == 0 else t1)

        @pl.when(cid == 1)
        def _():
            _stream_loop(tbl, gidx_v, sidx_v, bufs, acc, cnt_add=cadd,
                         table_hbm=t2 if phase == 0 else t3)

        plsc.subcore_barrier()

        _convert(acc, cnt_acc, tbl, r0b, ones_v, r0)
        pltpu.sync_copy(zrow, acc.at[pl.ds(r0, RPT)])
        pltpu.sync_copy(eg_h.at[sid], gidx_v)
        pltpu.sync_copy(vs_h.at[sid], sidx_v)
        plsc.subcore_barrier()

        _stream_loop(tbl, gidx_v, sidx_v, bufs, acc)
        plsc.subcore_barrier()

        @pl.when(cid == 0)
        def _():
            out = o0 if phase == 0 else o1
            pltpu.sync_copy(acc.at[pl.ds(r0, RPT)], out.at[pl.ds(r0, RPT)])

        @pl.when(cid == 1)
        def _():
            out = o2 if phase == 0 else o3
            pltpu.sync_copy(acc.at[pl.ds(r0, RPT)], out.at[pl.ds(r0, RPT)])

        if phase == 0:
            plsc.subcore_barrier()


_conv = functools.partial(
    pl.kernel,
    out_type=[jax.ShapeDtypeStruct((APAD, G), jnp.float32)] * 4,
    mesh=_MESH,
    compiler_params=pltpu.CompilerParams(use_tc_tiling_on_sc=False),
    scratch_types=[
        pltpu.VMEM((NCHUNK, B), jnp.int32),
        pltpu.VMEM((NCHUNK, B), jnp.int32),
        pltpu.VMEM((B, G), jnp.float32),
        pltpu.VMEM((B, G), jnp.float32),
        pltpu.VMEM((B, 16), jnp.float32),
        pltpu.VMEM_SHARED((APAD, G), jnp.float32),
        pltpu.VMEM_SHARED((APAD, G), jnp.float32),
        pltpu.VMEM_SHARED((APAD, 16), jnp.float32),
    ] + [pltpu.SemaphoreType.DMA] * 4,
)(_conv_body)


def _final_body(a0, a1, a2, a3, x0, x1, x2, x3, o_ref):
    o_ref[:, 0 * G:1 * G] = jnp.maximum(a0[...] + x0[...], 0.0)
    o_ref[:, 1 * G:2 * G] = jnp.maximum(a1[...] + x1[...], 0.0)
    o_ref[:, 2 * G:3 * G] = jnp.maximum(a2[...] + x2[...], 0.0)
    o_ref[:, 3 * G:4 * G] = jnp.maximum(a3[...] + x3[...], 0.0)


def _final(aggs, xts):
    return pl.pallas_call(
        _final_body,
        grid=(10,),
        in_specs=[pl.BlockSpec((1000, G), lambda i: (i, 0))] * 8,
        out_specs=pl.BlockSpec((1000, C), lambda i: (i, 0)),
        out_shape=jax.ShapeDtypeStruct((N, C), jnp.float32),
    )(*aggs, *xts)


def kernel(X, hyperedge_index, W, b):
    v = hyperedge_index[0].astype(jnp.int32)
    e = hyperedge_index[1].astype(jnp.int32)
    pad = NNZ_PAD - NNZ

    def _padded(idx, fill):
        p = jnp.concatenate([idx, jnp.full((pad,), fill, jnp.int32)])
        return p.reshape(NTILES, NCHUNK, B)

    v_g, v_s = _padded(v, 0), _padded(v, N)
    e_g, e_s = _padded(e, 0), _padded(e, N)

    zrow = jnp.zeros((RPT, G), jnp.float32)
    zcnt = jnp.zeros((RPT, 16), jnp.float32)
    ones = jnp.ones((B, 16), jnp.float32)

    xt = _matmul(X, W, b.reshape(1, C))
    aggs = _conv(*xt, v_g, e_s, e_g, v_s, zrow, zcnt, ones)
    return _final(aggs, xt)


# speedup vs baseline: 1.0797x; 1.0797x over previous
import functools

import jax
import jax.numpy as jnp
from jax import lax
from jax.experimental import pallas as pl
from jax.experimental.pallas import tpu as pltpu
from jax.experimental.pallas import tpu_sc as plsc

N = 10000
NNZ = 160000
C = 256
G = 64
NCORES = 2
NTILES = 16
B = 128
NCHUNK = 80
PER_TILE = NCHUNK * B
NNZ_PAD = NTILES * PER_TILE
APAD = 10112
RPT = APAD // NTILES
NBUF = 2

_MESH = plsc.VectorSubcoreMesh(
    core_axis_name="c", subcore_axis_name="s",
    num_cores=NCORES, num_subcores=NTILES)


def _mm_body(x_ref, w_ref, b_ref, o0_ref, o1_ref, o2_ref, o3_ref):
    acc = jnp.dot(x_ref[...], w_ref[...],
                  preferred_element_type=jnp.float32) + b_ref[...]
    o0_ref[...] = acc[:, 0 * G:1 * G]
    o1_ref[...] = acc[:, 1 * G:2 * G]
    o2_ref[...] = acc[:, 2 * G:3 * G]
    o3_ref[...] = acc[:, 3 * G:4 * G]


def _matmul(x, w, b2):
    return pl.pallas_call(
        _mm_body,
        grid=(10,),
        in_specs=[
            pl.BlockSpec((1000, C), lambda i: (i, 0)),
            pl.BlockSpec((C, C), lambda i: (0, 0)),
            pl.BlockSpec((1, C), lambda i: (0, 0)),
        ],
        out_specs=[pl.BlockSpec((1000, G), lambda i: (i, 0))] * 4,
        out_shape=[jax.ShapeDtypeStruct((N, G), jnp.float32)] * 4,
    )(x, w, b2)


def _stream_loop(table, gidx_v, sidx_v, bufs, acc, cnt_add=None,
                 table_hbm=None):
    def _issue(chunk, rows, gsem):
        if table_hbm is None:
            pltpu.async_copy(table.at[gidx_v.at[chunk]], rows, gsem)
        elif isinstance(chunk, int):
            src = table_hbm if chunk % 4 == 1 else table
            pltpu.async_copy(src.at[gidx_v.at[chunk]], rows, gsem)
        else:
            m = chunk % 4

            @pl.when(m == 1)
            def _():
                pltpu.async_copy(table_hbm.at[gidx_v.at[chunk]], rows, gsem)

            @pl.when(m != 1)
            def _():
                pltpu.async_copy(table.at[gidx_v.at[chunk]], rows, gsem)

    for b, (rows, gsem, ssem) in enumerate(bufs):
        _issue(b, rows, gsem)

    @pl.loop(0, NCHUNK, step=NBUF)
    def _(j):
        for b, (rows, gsem, ssem) in enumerate(bufs):
            jj = j + b
            pltpu.make_async_copy(table.at[gidx_v.at[jj]], rows, gsem).wait()
            pltpu.async_copy(rows, acc.at[sidx_v.at[jj]], ssem, add=True)
            if cnt_add is not None:
                ones_v, cnt_acc = cnt_add
                pltpu.async_copy(ones_v, cnt_acc.at[sidx_v.at[jj]], ssem,
                                 add=True)

            @pl.when(jj + NBUF < NCHUNK)
            def _():
                pltpu.make_async_copy(rows, acc.at[sidx_v.at[jj]], ssem).wait()
                if cnt_add is not None:
                    ones_v, cnt_acc = cnt_add
                    pltpu.make_async_copy(
                        ones_v, cnt_acc.at[sidx_v.at[jj]], ssem).wait()
                _issue(jj + NBUF, rows, gsem)

    for b, (rows, gsem, ssem) in enumerate(bufs):
        jj = NCHUNK - NBUF + b
        pltpu.make_async_copy(rows, acc.at[sidx_v.at[jj]], ssem).wait()
        if cnt_add is not None:
            ones_v, cnt_acc = cnt_add
            pltpu.make_async_copy(ones_v, cnt_acc.at[sidx_v.at[jj]],
                                  ssem).wait()


def _stage_tbl(src_hbm, tbl, sid):
    @pl.when(sid < 10)
    def _():
        off = sid * 1000
        pltpu.sync_copy(src_hbm.at[pl.ds(off, 1000)], tbl.at[pl.ds(off, 1000)])


def _convert(acc, cnt_acc, tbl, rows0, ones_v, r0):
    for off, rows_n in ((0, B), (B, B), (2 * B, B), (3 * B, B), (4 * B, 120)):
        base = r0 + off
        pltpu.sync_copy(acc.at[pl.ds(base, rows_n)],
                        rows0.at[pl.ds(0, rows_n)])
        pltpu.sync_copy(cnt_acc.at[pl.ds(base, rows_n)],
                        ones_v.at[pl.ds(0, rows_n)])

        @pl.loop(0, rows_n)
        def _(r):
            inv = 1.0 / jnp.maximum(ones_v[r], 1.0)
            for k in range(G // 16):
                sl = pl.ds(k * 16, 16)
                rows0[r, sl] = rows0[r, sl] * inv

        pltpu.sync_copy(rows0.at[pl.ds(0, rows_n)],
                        tbl.at[pl.ds(base, rows_n)])


def _conv_body(t0, t1, t2, t3, vg_h, es_h, eg_h, vs_h, zrow, zcnt, ones_h,
               o0, o1, o2, o3,
               gidx_v, sidx_v, r0b, r1b, ones_v, tbl, acc, cnt_acc,
               g0, g1, s0m, s1m):
    bufs = [(r0b, g0, s0m), (r1b, g1, s1m)]
    cid = lax.axis_index("c")
    sid = lax.axis_index("s")
    r0 = sid * RPT

    for phase in (0, 1):
        pltpu.sync_copy(zrow, acc.at[pl.ds(r0, RPT)])
        pltpu.sync_copy(vg_h.at[sid], gidx_v)
        pltpu.sync_copy(es_h.at[sid], sidx_v)
        if phase == 0:
            pltpu.sync_copy(zcnt, cnt_acc.at[pl.ds(r0, RPT)])
            pltpu.sync_copy(ones_h, ones_v)

        @pl.when(cid == 0)
        def _():
            _stage_tbl(t0 if phase == 0 else t1, tbl, sid)

        @pl.when(cid == 1)
        def _():
            _stage_tbl(t2 if phase == 0 else t3, tbl, sid)

        plsc.subcore_barrier()

        cadd = (ones_v, cnt_acc) if phase == 0 else None
        _stream_loop(tbl, gidx_v, sidx_v, bufs, acc, cnt_add=cadd)
        plsc.subcore_barrier()

        _convert(acc, cnt_acc, tbl, r0b, ones_v, r0)
        pltpu.sync_copy(zrow, acc.at[pl.ds(r0, RPT)])
        pltpu.sync_copy(eg_h.at[sid], gidx_v)
        pltpu.sync_copy(vs_h.at[sid], sidx_v)
        plsc.subcore_barrier()

        _stream_loop(tbl, gidx_v, sidx_v, bufs, acc)
        plsc.subcore_barrier()

        @pl.when(cid == 0)
        def _():
            out = o0 if phase == 0 else o1
            pltpu.sync_copy(acc.at[pl.ds(r0, RPT)], out.at[pl.ds(r0, RPT)])

        @pl.when(cid == 1)
        def _():
            out = o2 if phase == 0 else o3
            pltpu.sync_copy(acc.at[pl.ds(r0, RPT)], out.at[pl.ds(r0, RPT)])

        if phase == 0:
            plsc.subcore_barrier()


_conv = functools.partial(
    pl.kernel,
    out_type=[jax.ShapeDtypeStruct((APAD, G), jnp.float32)] * 4,
    mesh=_MESH,
    compiler_params=pltpu.CompilerParams(use_tc_tiling_on_sc=False),
    scratch_types=[
        pltpu.VMEM((NCHUNK, B), jnp.int32),
        pltpu.VMEM((NCHUNK, B), jnp.int32),
        pltpu.VMEM((B, G), jnp.float32),
        pltpu.VMEM((B, G), jnp.float32),
        pltpu.VMEM((B, 16), jnp.float32),
        pltpu.VMEM_SHARED((APAD, G), jnp.float32),
        pltpu.VMEM_SHARED((APAD, G), jnp.float32),
        pltpu.VMEM_SHARED((APAD, 16), jnp.float32),
    ] + [pltpu.SemaphoreType.DMA] * 4,
)(_conv_body)


def _final_body(a0, a1, a2, a3, x0, x1, x2, x3, o_ref):
    o_ref[:, 0 * G:1 * G] = jnp.maximum(a0[...] + x0[...], 0.0)
    o_ref[:, 1 * G:2 * G] = jnp.maximum(a1[...] + x1[...], 0.0)
    o_ref[:, 2 * G:3 * G] = jnp.maximum(a2[...] + x2[...], 0.0)
    o_ref[:, 3 * G:4 * G] = jnp.maximum(a3[...] + x3[...], 0.0)


def _final(aggs, xts):
    return pl.pallas_call(
        _final_body,
        grid=(10,),
        in_specs=[pl.BlockSpec((1000, G), lambda i: (i, 0))] * 8,
        out_specs=pl.BlockSpec((1000, C), lambda i: (i, 0)),
        out_shape=jax.ShapeDtypeStruct((N, C), jnp.float32),
    )(*aggs, *xts)


def kernel(X, hyperedge_index, W, b):
    v = hyperedge_index[0].astype(jnp.int32)
    e = hyperedge_index[1].astype(jnp.int32)
    pad = NNZ_PAD - NNZ

    def _padded(idx, fill):
        p = jnp.concatenate([idx, jnp.full((pad,), fill, jnp.int32)])
        return p.reshape(NTILES, NCHUNK, B)

    v_g, v_s = _padded(v, 0), _padded(v, N)
    e_g, e_s = _padded(e, 0), _padded(e, N)

    zrow = jnp.zeros((RPT, G), jnp.float32)
    zcnt = jnp.zeros((RPT, 16), jnp.float32)
    ones = jnp.ones((B, 16), jnp.float32)

    xt = _matmul(X, W, b.reshape(1, C))
    aggs = _conv(*xt, v_g, e_s, e_g, v_s, zrow, zcnt, ones)
    return _final(aggs, xt)


# speedup vs baseline: 1.1019x; 1.0206x over previous
import functools

import jax
import jax.numpy as jnp
from jax import lax
from jax.experimental import pallas as pl
from jax.experimental.pallas import tpu as pltpu
from jax.experimental.pallas import tpu_sc as plsc

N = 10000
NNZ = 160000
C = 256
G = 64
NCORES = 2
NTILES = 16
B = 128
NCHUNK = 80
PER_TILE = NCHUNK * B
NNZ_PAD = NTILES * PER_TILE
APAD = 10112
RPT = APAD // NTILES
NBUF = 2

_MESH = plsc.VectorSubcoreMesh(
    core_axis_name="c", subcore_axis_name="s",
    num_cores=NCORES, num_subcores=NTILES)


def _mm_body(x_ref, w_ref, b_ref, o0_ref, o1_ref, o2_ref, o3_ref):
    acc = jnp.dot(x_ref[...], w_ref[...],
                  preferred_element_type=jnp.float32) + b_ref[...]
    o0_ref[...] = acc[:, 0 * G:1 * G]
    o1_ref[...] = acc[:, 1 * G:2 * G]
    o2_ref[...] = acc[:, 2 * G:3 * G]
    o3_ref[...] = acc[:, 3 * G:4 * G]


def _matmul(x, w, b2):
    return pl.pallas_call(
        _mm_body,
        grid=(10,),
        in_specs=[
            pl.BlockSpec((1000, C), lambda i: (i, 0)),
            pl.BlockSpec((C, C), lambda i: (0, 0)),
            pl.BlockSpec((1, C), lambda i: (0, 0)),
        ],
        out_specs=[pl.BlockSpec((1000, G), lambda i: (i, 0))] * 4,
        out_shape=[jax.ShapeDtypeStruct((N, G), jnp.float32)] * 4,
    )(x, w, b2)


def _stream_loop(table, gidx_v, sidx_v, bufs, acc, cnt_add=None,
                 table_hbm=None):
    def _issue(chunk, rows, gsem):
        if table_hbm is None:
            pltpu.async_copy(table.at[gidx_v.at[chunk]], rows, gsem)
        elif isinstance(chunk, int):
            src = table_hbm if chunk % 4 == 1 else table
            pltpu.async_copy(src.at[gidx_v.at[chunk]], rows, gsem)
        else:
            m = chunk % 4

            @pl.when(m == 1)
            def _():
                pltpu.async_copy(table_hbm.at[gidx_v.at[chunk]], rows, gsem)

            @pl.when(m != 1)
            def _():
                pltpu.async_copy(table.at[gidx_v.at[chunk]], rows, gsem)

    for b, (rows, gsem, ssem) in enumerate(bufs):
        _issue(b, rows, gsem)

    @pl.loop(0, NCHUNK, step=NBUF)
    def _(j):
        for b, (rows, gsem, ssem) in enumerate(bufs):
            jj = j + b
            pltpu.make_async_copy(table.at[gidx_v.at[jj]], rows, gsem).wait()
            pltpu.async_copy(rows, acc.at[sidx_v.at[jj]], ssem, add=True)
            if cnt_add is not None:
                ones_v, cnt_acc = cnt_add
                pltpu.async_copy(ones_v, cnt_acc.at[sidx_v.at[jj]], ssem,
                                 add=True)

            @pl.when(jj + NBUF < NCHUNK)
            def _():
                pltpu.make_async_copy(rows, acc.at[sidx_v.at[jj]], ssem).wait()
                if cnt_add is not None:
                    ones_v, cnt_acc = cnt_add
                    pltpu.make_async_copy(
                        ones_v, cnt_acc.at[sidx_v.at[jj]], ssem).wait()
                _issue(jj + NBUF, rows, gsem)

    for b, (rows, gsem, ssem) in enumerate(bufs):
        jj = NCHUNK - NBUF + b
        pltpu.make_async_copy(rows, acc.at[sidx_v.at[jj]], ssem).wait()
        if cnt_add is not None:
            ones_v, cnt_acc = cnt_add
            pltpu.make_async_copy(ones_v, cnt_acc.at[sidx_v.at[jj]],
                                  ssem).wait()


def _stage_tbl(src_hbm, tbl, sid):
    @pl.when(sid < 10)
    def _():
        off = sid * 1000
        pltpu.sync_copy(src_hbm.at[pl.ds(off, 1000)], tbl.at[pl.ds(off, 1000)])


def _convert(acc, cnt_acc, tbl, rows0, ones_v, r0):
    for off, rows_n in ((0, B), (B, B), (2 * B, B), (3 * B, B), (4 * B, 120)):
        base = r0 + off
        pltpu.sync_copy(acc.at[pl.ds(base, rows_n)],
                        rows0.at[pl.ds(0, rows_n)])
        pltpu.sync_copy(cnt_acc.at[pl.ds(base, rows_n)],
                        ones_v.at[pl.ds(0, rows_n)])

        @pl.loop(0, rows_n)
        def _(r):
            inv = 1.0 / jnp.maximum(ones_v[r], 1.0)
            for k in range(G // 16):
                sl = pl.ds(k * 16, 16)
                rows0[r, sl] = rows0[r, sl] * inv

        pltpu.sync_copy(rows0.at[pl.ds(0, rows_n)],
                        tbl.at[pl.ds(base, rows_n)])


def _conv_body(t0, t1, t2, t3, vg_h, es_h, eg_h, vs_h, zrow, zcnt, ones_h,
               o0, o1, o2, o3,
               gidx_v, sidx_v, r0b, r1b, ones_v, tbl, acc, cnt_acc,
               g0, g1, s0m, s1m):
    bufs = [(r0b, g0, s0m), (r1b, g1, s1m)]
    cid = lax.axis_index("c")
    sid = lax.axis_index("s")
    r0 = sid * RPT

    for phase in (0, 1):
        pltpu.async_copy(zrow, acc.at[pl.ds(r0, RPT)], g0)
        pltpu.async_copy(vg_h.at[sid], gidx_v, g1)
        pltpu.async_copy(es_h.at[sid], sidx_v, s0m)
        if phase == 0:
            pltpu.async_copy(zcnt, cnt_acc.at[pl.ds(r0, RPT)], s1m)
            pltpu.async_copy(ones_h, ones_v, s1m)

        off = sid * 1000

        @pl.when(jnp.logical_and(cid == 0, sid < 10))
        def _():
            src = t0 if phase == 0 else t1
            pltpu.async_copy(src.at[pl.ds(off, 1000)],
                             tbl.at[pl.ds(off, 1000)], g0)

        @pl.when(jnp.logical_and(cid == 1, sid < 10))
        def _():
            src = t2 if phase == 0 else t3
            pltpu.async_copy(src.at[pl.ds(off, 1000)],
                             tbl.at[pl.ds(off, 1000)], g0)

        pltpu.make_async_copy(zrow, acc.at[pl.ds(r0, RPT)], g0).wait()
        pltpu.make_async_copy(vg_h.at[sid], gidx_v, g1).wait()
        pltpu.make_async_copy(es_h.at[sid], sidx_v, s0m).wait()
        if phase == 0:
            pltpu.make_async_copy(zcnt, cnt_acc.at[pl.ds(r0, RPT)],
                                  s1m).wait()
            pltpu.make_async_copy(ones_h, ones_v, s1m).wait()

        @pl.when(sid < 10)
        def _():
            pltpu.make_async_copy(t0.at[pl.ds(off, 1000)],
                                  tbl.at[pl.ds(off, 1000)], g0).wait()

        plsc.subcore_barrier()

        cadd = (ones_v, cnt_acc) if phase == 0 else None
        _stream_loop(tbl, gidx_v, sidx_v, bufs, acc, cnt_add=cadd)
        plsc.subcore_barrier()

        pltpu.async_copy(eg_h.at[sid], gidx_v, g1)
        pltpu.async_copy(vs_h.at[sid], sidx_v, s0m)
        _convert(acc, cnt_acc, tbl, r0b, ones_v, r0)
        pltpu.async_copy(zrow, acc.at[pl.ds(r0, RPT)], g0)
        pltpu.make_async_copy(zrow, acc.at[pl.ds(r0, RPT)], g0).wait()
        pltpu.make_async_copy(eg_h.at[sid], gidx_v, g1).wait()
        pltpu.make_async_copy(vs_h.at[sid], sidx_v, s0m).wait()
        plsc.subcore_barrier()

        _stream_loop(tbl, gidx_v, sidx_v, bufs, acc)
        plsc.subcore_barrier()

        @pl.when(cid == 0)
        def _():
            out = o0 if phase == 0 else o1
            pltpu.sync_copy(acc.at[pl.ds(r0, RPT)], out.at[pl.ds(r0, RPT)])

        @pl.when(cid == 1)
        def _():
            out = o2 if phase == 0 else o3
            pltpu.sync_copy(acc.at[pl.ds(r0, RPT)], out.at[pl.ds(r0, RPT)])

        if phase == 0:
            plsc.subcore_barrier()


_conv = functools.partial(
    pl.kernel,
    out_type=[jax.ShapeDtypeStruct((APAD, G), jnp.float32)] * 4,
    mesh=_MESH,
    compiler_params=pltpu.CompilerParams(use_tc_tiling_on_sc=False),
    scratch_types=[
        pltpu.VMEM((NCHUNK, B), jnp.int32),
        pltpu.VMEM((NCHUNK, B), jnp.int32),
        pltpu.VMEM((B, G), jnp.float32),
        pltpu.VMEM((B, G), jnp.float32),
        pltpu.VMEM((B, 16), jnp.float32),
        pltpu.VMEM_SHARED((APAD, G), jnp.float32),
        pltpu.VMEM_SHARED((APAD, G), jnp.float32),
        pltpu.VMEM_SHARED((APAD, 16), jnp.float32),
    ] + [pltpu.SemaphoreType.DMA] * 4,
)(_conv_body)


def _final_body(a0, a1, a2, a3, x0, x1, x2, x3, o_ref):
    o_ref[:, 0 * G:1 * G] = jnp.maximum(a0[...] + x0[...], 0.0)
    o_ref[:, 1 * G:2 * G] = jnp.maximum(a1[...] + x1[...], 0.0)
    o_ref[:, 2 * G:3 * G] = jnp.maximum(a2[...] + x2[...], 0.0)
    o_ref[:, 3 * G:4 * G] = jnp.maximum(a3[...] + x3[...], 0.0)


def _final(aggs, xts):
    return pl.pallas_call(
        _final_body,
        grid=(10,),
        in_specs=[pl.BlockSpec((1000, G), lambda i: (i, 0))] * 8,
        out_specs=pl.BlockSpec((1000, C), lambda i: (i, 0)),
        out_shape=jax.ShapeDtypeStruct((N, C), jnp.float32),
    )(*aggs, *xts)


def kernel(X, hyperedge_index, W, b):
    v = hyperedge_index[0].astype(jnp.int32)
    e = hyperedge_index[1].astype(jnp.int32)
    pad = NNZ_PAD - NNZ

    def _padded(idx, fill):
        p = jnp.concatenate([idx, jnp.full((pad,), fill, jnp.int32)])
        return p.reshape(NTILES, NCHUNK, B)

    v_g, v_s = _padded(v, 0), _padded(v, N)
    e_g, e_s = _padded(e, 0), _padded(e, N)

    zrow = jnp.zeros((RPT, G), jnp.float32)
    zcnt = jnp.zeros((RPT, 16), jnp.float32)
    ones = jnp.ones((B, 16), jnp.float32)

    xt = _matmul(X, W, b.reshape(1, C))
    aggs = _conv(*xt, v_g, e_s, e_g, v_s, zrow, zcnt, ones)
    return _final(aggs, xt)


# speedup vs baseline: 1.1023x; 1.0004x over previous
import functools

import jax
import jax.numpy as jnp
from jax import lax
from jax.experimental import pallas as pl
from jax.experimental.pallas import tpu as pltpu
from jax.experimental.pallas import tpu_sc as plsc

N = 10000
NNZ = 160000
C = 256
G = 64
NCORES = 2
NTILES = 16
B = 128
NCHUNK = 80
PER_TILE = NCHUNK * B
NNZ_PAD = NTILES * PER_TILE
APAD = 10112
RPT = APAD // NTILES
NBUF = 2

_MESH = plsc.VectorSubcoreMesh(
    core_axis_name="c", subcore_axis_name="s",
    num_cores=NCORES, num_subcores=NTILES)


def _mm_body(x_ref, w_ref, b_ref, o0_ref, o1_ref, o2_ref, o3_ref):
    acc = jnp.dot(x_ref[...], w_ref[...],
                  preferred_element_type=jnp.float32) + b_ref[...]
    o0_ref[...] = acc[:, 0 * G:1 * G]
    o1_ref[...] = acc[:, 1 * G:2 * G]
    o2_ref[...] = acc[:, 2 * G:3 * G]
    o3_ref[...] = acc[:, 3 * G:4 * G]


def _matmul(x, w, b2):
    return pl.pallas_call(
        _mm_body,
        grid=(10,),
        in_specs=[
            pl.BlockSpec((1000, C), lambda i: (i, 0)),
            pl.BlockSpec((C, C), lambda i: (0, 0)),
            pl.BlockSpec((1, C), lambda i: (0, 0)),
        ],
        out_specs=[pl.BlockSpec((1000, G), lambda i: (i, 0))] * 4,
        out_shape=[jax.ShapeDtypeStruct((N, G), jnp.float32)] * 4,
    )(x, w, b2)


def _stream_loop(table, gidx_v, sidx_v, bufs, acc, cnt_add=None):
    def _issue(chunk, rows, gsem):
        pltpu.async_copy(table.at[gidx_v.at[chunk]], rows, gsem)

    for b, (rows, gsem, ssem) in enumerate(bufs):
        _issue(b, rows, gsem)

    @pl.loop(0, NCHUNK, step=NBUF)
    def _(j):
        for b, (rows, gsem, ssem) in enumerate(bufs):
            jj = j + b
            pltpu.make_async_copy(table.at[gidx_v.at[jj]], rows, gsem).wait()
            pltpu.async_copy(rows, acc.at[sidx_v.at[jj]], ssem, add=True)
            if cnt_add is not None:
                ones_v, cnt_acc = cnt_add
                pltpu.async_copy(ones_v, cnt_acc.at[sidx_v.at[jj]], ssem,
                                 add=True)

            @pl.when(jj + NBUF < NCHUNK)
            def _():
                pltpu.make_async_copy(rows, acc.at[sidx_v.at[jj]], ssem).wait()
                if cnt_add is not None:
                    ones_v, cnt_acc = cnt_add
                    pltpu.make_async_copy(
                        ones_v, cnt_acc.at[sidx_v.at[jj]], ssem).wait()
                _issue(jj + NBUF, rows, gsem)

    for b, (rows, gsem, ssem) in enumerate(bufs):
        jj = NCHUNK - NBUF + b
        pltpu.make_async_copy(rows, acc.at[sidx_v.at[jj]], ssem).wait()
        if cnt_add is not None:
            ones_v, cnt_acc = cnt_add
            pltpu.make_async_copy(ones_v, cnt_acc.at[sidx_v.at[jj]],
                                  ssem).wait()


def _convert(acc, cnt_acc, tbl, rows0, ones_v, r0):
    for off, rows_n in ((0, B), (B, B), (2 * B, B), (3 * B, B), (4 * B, 120)):
        base = r0 + off
        pltpu.sync_copy(acc.at[pl.ds(base, rows_n)],
                        rows0.at[pl.ds(0, rows_n)])
        pltpu.sync_copy(cnt_acc.at[pl.ds(base, rows_n)],
                        ones_v.at[pl.ds(0, rows_n)])

        @pl.loop(0, rows_n)
        def _(r):
            inv = 1.0 / jnp.maximum(ones_v[r], 1.0)
            for k in range(G // 16):
                sl = pl.ds(k * 16, 16)
                rows0[r, sl] = rows0[r, sl] * inv

        pltpu.sync_copy(rows0.at[pl.ds(0, rows_n)],
                        tbl.at[pl.ds(base, rows_n)])


def _conv_body(t0, t1, t2, t3, vg_h, es_h, eg_h, vs_h, zrow, zcnt, ones_h,
               o0, o1, o2, o3,
               gidx_v, sidx_v, r0b, r1b, ones_v, tbl, acc, cnt_acc,
               g0, g1, s0m, s1m):
    bufs = [(r0b, g0, s0m), (r1b, g1, s1m)]
    cid = lax.axis_index("c")
    sid = lax.axis_index("s")
    r0 = sid * RPT

    for phase in (0, 1):
        pltpu.async_copy(zrow, acc.at[pl.ds(r0, RPT)], g0)
        pltpu.async_copy(vg_h.at[sid], gidx_v, g1)
        pltpu.async_copy(es_h.at[sid], sidx_v, s0m)
        if phase == 0:
            pltpu.async_copy(zcnt, cnt_acc.at[pl.ds(r0, RPT)], s1m)
            pltpu.async_copy(ones_h, ones_v, s1m)

        off = sid * 1000

        @pl.when(jnp.logical_and(cid == 0, sid < 10))
        def _():
            src = t0 if phase == 0 else t1
            pltpu.async_copy(src.at[pl.ds(off, 1000)],
                             tbl.at[pl.ds(off, 1000)], g0)

        @pl.when(jnp.logical_and(cid == 1, sid < 10))
        def _():
            src = t2 if phase == 0 else t3
            pltpu.async_copy(src.at[pl.ds(off, 1000)],
                             tbl.at[pl.ds(off, 1000)], g0)

        pltpu.make_async_copy(zrow, acc.at[pl.ds(r0, RPT)], g0).wait()
        pltpu.make_async_copy(vg_h.at[sid], gidx_v, g1).wait()
        pltpu.make_async_copy(es_h.at[sid], sidx_v, s0m).wait()
        if phase == 0:
            pltpu.make_async_copy(zcnt, cnt_acc.at[pl.ds(r0, RPT)],
                                  s1m).wait()
            pltpu.make_async_copy(ones_h, ones_v, s1m).wait()

        @pl.when(sid < 10)
        def _():
            pltpu.make_async_copy(t0.at[pl.ds(off, 1000)],
                                  tbl.at[pl.ds(off, 1000)], g0).wait()

        plsc.subcore_barrier()

        cadd = (ones_v, cnt_acc) if phase == 0 else None
        _stream_loop(tbl, gidx_v, sidx_v, bufs, acc, cnt_add=cadd)
        plsc.subcore_barrier()

        pltpu.async_copy(eg_h.at[sid], gidx_v, g1)
        pltpu.async_copy(vs_h.at[sid], sidx_v, s0m)
        _convert(acc, cnt_acc, tbl, r0b, ones_v, r0)
        pltpu.async_copy(zrow, acc.at[pl.ds(r0, RPT)], g0)
        pltpu.make_async_copy(zrow, acc.at[pl.ds(r0, RPT)], g0).wait()
        pltpu.make_async_copy(eg_h.at[sid], gidx_v, g1).wait()
        pltpu.make_async_copy(vs_h.at[sid], sidx_v, s0m).wait()
        plsc.subcore_barrier()

        _stream_loop(tbl, gidx_v, sidx_v, bufs, acc)
        plsc.subcore_barrier()

        @pl.when(cid == 0)
        def _():
            out = o0 if phase == 0 else o1
            pltpu.sync_copy(acc.at[pl.ds(r0, RPT)], out.at[pl.ds(r0, RPT)])

        @pl.when(cid == 1)
        def _():
            out = o2 if phase == 0 else o3
            pltpu.sync_copy(acc.at[pl.ds(r0, RPT)], out.at[pl.ds(r0, RPT)])

        if phase == 0:
            plsc.subcore_barrier()


_conv = functools.partial(
    pl.kernel,
    out_type=[jax.ShapeDtypeStruct((APAD, G), jnp.float32)] * 4,
    mesh=_MESH,
    compiler_params=pltpu.CompilerParams(use_tc_tiling_on_sc=False),
    scratch_types=[
        pltpu.VMEM((NCHUNK, B), jnp.int32),
        pltpu.VMEM((NCHUNK, B), jnp.int32),
        pltpu.VMEM((B, G), jnp.float32),
        pltpu.VMEM((B, G), jnp.float32),
        pltpu.VMEM((B, 16), jnp.float32),
        pltpu.VMEM_SHARED((APAD, G), jnp.float32),
        pltpu.VMEM_SHARED((APAD, G), jnp.float32),
        pltpu.VMEM_SHARED((APAD, 16), jnp.float32),
    ] + [pltpu.SemaphoreType.DMA] * 4,
)(_conv_body)


def _final_body(a0, a1, a2, a3, x0, x1, x2, x3, o_ref):
    o_ref[:, 0 * G:1 * G] = jnp.maximum(a0[...] + x0[...], 0.0)
    o_ref[:, 1 * G:2 * G] = jnp.maximum(a1[...] + x1[...], 0.0)
    o_ref[:, 2 * G:3 * G] = jnp.maximum(a2[...] + x2[...], 0.0)
    o_ref[:, 3 * G:4 * G] = jnp.maximum(a3[...] + x3[...], 0.0)


def _final(aggs, xts):
    return pl.pallas_call(
        _final_body,
        grid=(10,),
        in_specs=[pl.BlockSpec((1000, G), lambda i: (i, 0))] * 8,
        out_specs=pl.BlockSpec((1000, C), lambda i: (i, 0)),
        out_shape=jax.ShapeDtypeStruct((N, C), jnp.float32),
    )(*aggs, *xts)


def kernel(X, hyperedge_index, W, b):
    v = hyperedge_index[0].astype(jnp.int32)
    e = hyperedge_index[1].astype(jnp.int32)
    pad = NNZ_PAD - NNZ

    def _padded(idx, fill):
        p = jnp.concatenate([idx, jnp.full((pad,), fill, jnp.int32)])
        return p.reshape(NTILES, NCHUNK, B)

    v_g, v_s = _padded(v, 0), _padded(v, N)
    e_g, e_s = _padded(e, 0), _padded(e, N)

    zrow = jnp.zeros((RPT, G), jnp.float32)
    zcnt = jnp.zeros((RPT, 16), jnp.float32)
    ones = jnp.ones((B, 16), jnp.float32)

    xt = _matmul(X, W, b.reshape(1, C))
    aggs = _conv(*xt, v_g, e_s, e_g, v_s, zrow, zcnt, ones)
    return _final(aggs, xt)
